# baseline (speedup 1.0000x reference)
import functools

import jax
import jax.numpy as jnp
from jax import lax
from jax.experimental import pallas as pl
from jax.experimental.pallas import tpu as pltpu
from jax.experimental.pallas import tpu_sc as plsc

N = 10000
D = 128
E = 320000

NC = 2
NS = 16
DH = D // NC
C = 128
CH = 160
E_PAD = NS * CH * C
ROWS_A = 624


def _mm_body(feat_ref, w_ref, out_ref):
    r = jnp.dot(feat_ref[...], w_ref[...], preferred_element_type=jnp.float32)
    out_ref[0] = r[:, :DH].astype(jnp.bfloat16)
    out_ref[1] = r[:, DH:].astype(jnp.bfloat16)


def _support_matmul(feat, W):
    BLK = 1000
    return pl.pallas_call(
        _mm_body,
        grid=(N // BLK,),
        in_specs=[
            pl.BlockSpec((BLK, D), lambda i: (i, 0)),
            pl.BlockSpec((D, D), lambda i: (0, 0)),
        ],
        out_specs=pl.BlockSpec((NC, BLK, DH), lambda i: (0, i, 0)),
        out_shape=jax.ShapeDtypeStruct((NC, N, DH), jnp.bfloat16),
    )(feat, W)


def _sc_body(sup_hbm, srcb_hbm, dstb_hbm, ewb_hbm, zeros_hbm, out_hbm,
             src_v, dst_v, rows0, rows1, rows2, rows3,
             rowsq0, rowsq1, rowsq2, rowsq3, ew_v, acc,
             semr0, semr1, semr2, semr3, semw0, semw1, semw2, semw3):
    cid = lax.axis_index("c")
    sid = lax.axis_index("s")

    pltpu.sync_copy(srcb_hbm.at[sid], src_v)
    pltpu.sync_copy(dstb_hbm.at[sid], dst_v)
    pltpu.sync_copy(ewb_hbm.at[sid], ew_v)

    pltpu.sync_copy(zeros_hbm, acc.at[pl.ds(sid * ROWS_A, ROWS_A)])

    @pl.when(sid == NS - 1)
    def _():
        pltpu.sync_copy(zeros_hbm.at[pl.ds(0, 16)], acc.at[pl.ds(NS * ROWS_A, 16)])

    plsc.subcore_barrier()

    sup = sup_hbm.at[cid]
    rows = (rows0, rows1, rows2, rows3)
    rowsq = (rowsq0, rowsq1, rowsq2, rowsq3)
    semr = (semr0, semr1, semr2, semr3)
    semw = (semw0, semw1, semw2, semw3)
    NB = 4

    def _issue(kk, b):
        pltpu.async_copy(sup.at[src_v.at[kk]], rows[b], semr[b])

    for b in range(NB):
        _issue(b, b)

    def _scale_group(g, carry, b, kk):
        w16 = ew_v[kk, pl.ds(g * 16, 16)]
        for u in range(16):
            wb = lax.gather(
                w16, jnp.full((16, 1), u, jnp.int32),
                lax.GatherDimensionNumbers(
                    offset_dims=(), collapsed_slice_dims=(0,),
                    start_index_map=(0,)),
                (1,), mode=lax.GatherScatterMode.PROMISE_IN_BOUNDS)
            wb32 = plsc.pack(wb, wb, format=plsc.PackFormat.INTERLEAVED)
            e = g * 16 + u
            for fb in range(DH // 32):
                prod = rows[b][e, pl.ds(fb * 32, 32)] * wb32
                rowsq[b][e, pl.ds(fb * 32, 32)] = prod.astype(jnp.int16)
        return carry

    def _outer(i, carry):
        k = i * NB
        for b in range(NB):
            kk = k + b
            pltpu.make_async_copy(sup.at[src_v.at[kk]], rows[b], semr[b]).wait()

            @pl.when(kk >= NB)
            def _():
                pltpu.make_async_copy(
                    rowsq[b], acc.at[dst_v.at[kk - NB]], semw[b]).wait()

            lax.fori_loop(0, C // 16,
                          functools.partial(_scale_group, b=b, kk=kk), 0)
            pltpu.async_copy(rowsq[b], acc.at[dst_v.at[kk]], semw[b], add=True)

            @pl.when(kk + NB < CH)
            def _():
                _issue(kk + NB, b)
        return carry

    lax.fori_loop(0, CH // NB, _outer, 0)
    for b in range(NB):
        pltpu.make_async_copy(rowsq[b], acc.at[dst_v.at[CH - NB + b]],
                              semw[b]).wait()
    plsc.subcore_barrier()

    sl = pl.ds(sid * ROWS_A, ROWS_A)
    pltpu.sync_copy(acc.at[sl], out_hbm.at[cid, sl])

    @pl.when(sid == NS - 1)
    def _():
        tl = pl.ds(NS * ROWS_A, 16)
        pltpu.sync_copy(acc.at[tl], out_hbm.at[cid, tl])


def _sc_aggregate(support, srcb, dstb, ewb, zeros):
    mesh = plsc.VectorSubcoreMesh(core_axis_name="c", subcore_axis_name="s")
    f = pl.kernel(
        _sc_body,
        out_type=jax.ShapeDtypeStruct((NC, N, DH), jnp.int16),
        mesh=mesh,
        compiler_params=pltpu.CompilerParams(use_tc_tiling_on_sc=False,
                                             needs_layout_passes=False),
        scratch_types=[
            pltpu.VMEM((CH, C), jnp.int32),
            pltpu.VMEM((CH, C), jnp.int32),
            pltpu.VMEM((C, DH), jnp.bfloat16),
            pltpu.VMEM((C, DH), jnp.bfloat16),
            pltpu.VMEM((C, DH), jnp.bfloat16),
            pltpu.VMEM((C, DH), jnp.bfloat16),
            pltpu.VMEM((C, DH), jnp.int16),
            pltpu.VMEM((C, DH), jnp.int16),
            pltpu.VMEM((C, DH), jnp.int16),
            pltpu.VMEM((C, DH), jnp.int16),
            pltpu.VMEM((CH, C), jnp.float32),
            pltpu.VMEM_SHARED((N, DH), jnp.int16),
        ] + [pltpu.SemaphoreType.DMA] * 8,
    )
    return f(support, srcb, dstb, ewb, zeros)


def _fin_body(p_ref, b_ref, out_ref):
    full = jnp.concatenate([p_ref[0], p_ref[1]], axis=1).astype(jnp.float32)
    out_ref[...] = jnp.maximum(full * (1.0 / 256.0) + b_ref[...], 0.0)


def _finalize(partials, b):
    BLK = 1000
    return pl.pallas_call(
        _fin_body,
        grid=(N // BLK,),
        in_specs=[
            pl.BlockSpec((NC, BLK, DH), lambda i: (0, i, 0)),
            pl.BlockSpec((D,), lambda i: (0,)),
        ],
        out_specs=pl.BlockSpec((BLK, D), lambda i: (i, 0)),
        out_shape=jax.ShapeDtypeStruct((N, D), jnp.float32),
    )(partials, b)


def kernel(feat, edge_index, edge_weight, W, b):
    support = _support_matmul(feat, W)

    src = edge_index[0].astype(jnp.int32)
    dst = edge_index[1].astype(jnp.int32)
    ew = edge_weight.astype(jnp.float32)

    pad = E_PAD - E
    srcb = jnp.pad(src, (0, pad)).reshape(NS, CH, C)
    dstb = jnp.pad(dst, (0, pad)).reshape(NS, CH, C)
    ewb = (jnp.pad(ew, (0, pad)) * 256.0).reshape(NS, CH, C)
    zeros = jnp.zeros((ROWS_A, DH), jnp.int16)

    partials = _sc_aggregate(support, srcb, dstb, ewb, zeros)
    return _finalize(partials, b)
